# baseline (speedup 1.0000x reference)
import functools

import jax
import jax.numpy as jnp
from jax import lax
from jax.experimental import pallas as pl
from jax.experimental.pallas import tpu as pltpu
from jax.experimental.pallas import tpu_sc as plsc

NF = 26
V = 100000
D = 32
B = 16384
NCONT = 13
OUTW = NF * D + NCONT
TW = 128
TR = NF * V // 4
NFH = NF // 2

NC, NS = 2, 16
NW = NC * NS
RPW = B // NW
C = 16
NCH = RPW // C

_mesh = plsc.VectorSubcoreMesh(
    core_axis_name="c", subcore_axis_name="s", num_cores=NC, num_subcores=NS
)


@functools.partial(
    pl.kernel,
    out_type=jax.ShapeDtypeStruct((B * OUTW,), jnp.float32),
    mesh=_mesh,
    scratch_types=[
        pltpu.VMEM((C, NF), jnp.int32),
        pltpu.VMEM((NFH * C, TW), jnp.float32),
        pltpu.VMEM((NFH * C, TW), jnp.float32),
        pltpu.VMEM((C * NCONT,), jnp.float32),
        pltpu.VMEM((C * OUTW,), jnp.float32),
        pltpu.SemaphoreType.DMA,
        pltpu.SemaphoreType.DMA,
        pltpu.SemaphoreType.DMA,
    ],
    compiler_params=pltpu.CompilerParams(needs_layout_passes=False),
)
def _emb_gather(table_hbm, cat_hbm, cont_hbm, out_hbm,
                catb, gbuf0, gbuf1, cbuf, obuf, sem0, sem1, semc):
    wid = lax.axis_index("s") * NC + lax.axis_index("c")
    base = wid * RPW
    lane = jnp.arange(16, dtype=jnp.int32)
    lane_out = lane * OUTW
    lane_cont = lane * NCONT

    def quad_idx(f):
        ids = plsc.load_gather(catb, [lane, jnp.full((16,), f, jnp.int32)])
        return (ids + f * V) >> 2

    def sub_off(f):
        ids = plsc.load_gather(catb, [lane, jnp.full((16,), f, jnp.int32)])
        return ((ids + f * V) & 3) << 5

    def fire(half, gbuf, sem):
        return [
            pltpu.async_copy(
                table_hbm.at[quad_idx(half * NFH + f)],
                gbuf.at[pl.ds(f * C, C)],
                sem,
            )
            for f in range(NFH)
        ]

    def merge(half, gbuf):
        def field(f, carry2):
            fa = half * NFH + f
            offv = sub_off(fa)
            rows = f * C + lane
            dst0 = lane_out + fa * D
            for e in range(D):
                v = plsc.load_gather(gbuf, [rows, offv + e])
                plsc.store_scatter(obuf, [dst0 + e], v)
            return carry2

        lax.fori_loop(0, NFH, field, 0)

    def chunk(c, carry):
        r0 = base + c * C
        pltpu.sync_copy(cat_hbm.at[pl.ds(r0, C)], catb)
        cps0 = fire(0, gbuf0, sem0)
        cps1 = fire(1, gbuf1, sem1)
        cpc = pltpu.async_copy(
            cont_hbm.at[pl.ds(r0 * NCONT, C * NCONT)], cbuf, semc
        )

        cpc.wait()
        for e in range(NCONT):
            v = plsc.load_gather(cbuf, [lane_cont + e])
            plsc.store_scatter(obuf, [lane_out + (NF * D + e)], v)

        for cp in cps0:
            cp.wait()
        merge(0, gbuf0)
        for cp in cps1:
            cp.wait()
        merge(1, gbuf1)

        pltpu.sync_copy(obuf, out_hbm.at[pl.ds(r0 * OUTW, C * OUTW)])
        return carry

    lax.fori_loop(0, NCH, chunk, 0)


def kernel(cont_data, cat_data, tables):
    zero = cont_data[0, 0] * jnp.float32(0.0)
    table_q = tables.reshape(TR, TW) + zero
    out = _emb_gather(
        table_q,
        cat_data,
        cont_data.reshape(B * NCONT),
    )
    return out.reshape(B, OUTW)


# speedup vs baseline: 1.1182x; 1.1182x over previous
import functools

import jax
import jax.numpy as jnp
from jax import lax
from jax.experimental import pallas as pl
from jax.experimental.pallas import tpu as pltpu
from jax.experimental.pallas import tpu_sc as plsc

NF = 26
V = 100000
D = 32
B = 16384
NCONT = 13
OUTW = NF * D + NCONT
TW = 128
TR = NF * V // 4
NFH = NF // 2

NC, NS = 2, 16
NW = NC * NS
RPW = B // NW
C = 16
NCH = RPW // C

_mesh = plsc.VectorSubcoreMesh(
    core_axis_name="c", subcore_axis_name="s", num_cores=NC, num_subcores=NS
)


@functools.partial(
    pl.kernel,
    out_type=jax.ShapeDtypeStruct((B * OUTW,), jnp.float32),
    mesh=_mesh,
    scratch_types=[
        pltpu.VMEM((C, NF), jnp.int32),
        pltpu.VMEM((NFH * C, TW), jnp.float32),
        pltpu.VMEM((NFH * C, TW), jnp.float32),
        pltpu.VMEM((C * NCONT,), jnp.float32),
        pltpu.VMEM((C * OUTW,), jnp.float32),
        pltpu.SemaphoreType.DMA,
        pltpu.SemaphoreType.DMA,
        pltpu.SemaphoreType.DMA,
    ],
    compiler_params=pltpu.CompilerParams(needs_layout_passes=False),
)
def _emb_gather(table_hbm, cat_hbm, cont_hbm, out_hbm,
                catb, gbuf0, gbuf1, cbuf, obuf, sem0, sem1, semc):
    wid = lax.axis_index("s") * NC + lax.axis_index("c")
    base = wid * RPW
    lane = jnp.arange(16, dtype=jnp.int32)
    lane_out = lane * OUTW
    lane_cont = lane * NCONT

    def quad_idx(f):
        ids = plsc.load_gather(catb, [lane, jnp.full((16,), f, jnp.int32)])
        return (ids + f * V) >> 2

    def sub_off(f):
        ids = plsc.load_gather(catb, [lane, jnp.full((16,), f, jnp.int32)])
        return ((ids + f * V) & 3) << 5

    def fire(half, gbuf, sem):
        return [
            pltpu.async_copy(
                table_hbm.at[quad_idx(half * NFH + f)],
                gbuf.at[pl.ds(f * C, C)],
                sem,
            )
            for f in range(NFH)
        ]

    def merge(half, gbuf):
        def field(f, carry2):
            fa = half * NFH + f
            offv = sub_off(fa)
            rows = f * C + lane
            dst0 = lane_out + fa * D
            for e in range(D):
                v = plsc.load_gather(gbuf, [rows, offv + e])
                plsc.store_scatter(obuf, [dst0 + e], v)
            return carry2

        lax.fori_loop(0, NFH, field, 0)

    def chunk(c, carry):
        r0 = base + c * C
        pltpu.sync_copy(cat_hbm.at[pl.ds(r0, C)], catb)
        cps0 = fire(0, gbuf0, sem0)
        cps1 = fire(1, gbuf1, sem1)
        cpc = pltpu.async_copy(
            cont_hbm.at[pl.ds(r0 * NCONT, C * NCONT)], cbuf, semc
        )

        cpc.wait()
        for e in range(NCONT):
            v = plsc.load_gather(cbuf, [lane_cont + e])
            plsc.store_scatter(obuf, [lane_out + (NF * D + e)], v)

        for cp in cps0:
            cp.wait()
        merge(0, gbuf0)
        for cp in cps1:
            cp.wait()
        merge(1, gbuf1)

        pltpu.sync_copy(obuf, out_hbm.at[pl.ds(r0 * OUTW, C * OUTW)])
        return carry

    lax.fori_loop(0, NCH, chunk, 0)


def kernel(cont_data, cat_data, tables):
    out = _emb_gather(
        tables.reshape(TR, TW),
        cat_data,
        cont_data.reshape(B * NCONT),
    )
    return out.reshape(B, OUTW)


# speedup vs baseline: 1.1259x; 1.0069x over previous
import functools

import jax
import jax.numpy as jnp
from jax import lax
from jax.experimental import pallas as pl
from jax.experimental.pallas import tpu as pltpu
from jax.experimental.pallas import tpu_sc as plsc

NF = 26
V = 100000
D = 32
B = 16384
NCONT = 13
OUTW = NF * D + NCONT
TW = 128
TR = NF * V // 4
NFH = NF // 2

NC, NS = 2, 16
NW = NC * NS
RPW = B // NW
C = 16
NCH = RPW // C

_mesh = plsc.VectorSubcoreMesh(
    core_axis_name="c", subcore_axis_name="s", num_cores=NC, num_subcores=NS
)


@functools.partial(
    pl.kernel,
    out_type=jax.ShapeDtypeStruct((B * OUTW,), jnp.float32),
    mesh=_mesh,
    scratch_types=[
        pltpu.VMEM((C, NF), jnp.int32),
        pltpu.VMEM((NFH * C, TW), jnp.float32),
        pltpu.VMEM((NFH * C, TW), jnp.float32),
        pltpu.VMEM((C * NCONT,), jnp.float32),
        pltpu.VMEM((C * OUTW,), jnp.float32),
        pltpu.VMEM((C * OUTW,), jnp.float32),
        pltpu.SemaphoreType.DMA,
        pltpu.SemaphoreType.DMA,
        pltpu.SemaphoreType.DMA,
        pltpu.SemaphoreType.DMA,
    ],
    compiler_params=pltpu.CompilerParams(needs_layout_passes=False),
)
def _emb_gather(table_hbm, cat_hbm, cont_hbm, out_hbm,
                catb, gbuf0, gbuf1, cbuf, obufa, obufb, sem0, sem1, semc, semw):
    wid = lax.axis_index("s") * NC + lax.axis_index("c")
    base = wid * RPW
    lane = jnp.arange(16, dtype=jnp.int32)
    lane_out = lane * OUTW
    lane_cont = lane * NCONT

    def quad_idx(f):
        ids = plsc.load_gather(catb, [lane, jnp.full((16,), f, jnp.int32)])
        return (ids + f * V) >> 2

    def sub_off(f):
        ids = plsc.load_gather(catb, [lane, jnp.full((16,), f, jnp.int32)])
        return ((ids + f * V) & 3) << 5

    def fire(half, gbuf, sem):
        return [
            pltpu.async_copy(
                table_hbm.at[quad_idx(half * NFH + f)],
                gbuf.at[pl.ds(f * C, C)],
                sem,
            )
            for f in range(NFH)
        ]

    def merge(half, gbuf, obuf):
        def field(f, carry2):
            fa = half * NFH + f
            offv = sub_off(fa)
            rows = f * C + lane
            dst0 = lane_out + fa * D
            for e in range(D):
                v = plsc.load_gather(gbuf, [rows, offv + e])
                plsc.store_scatter(obuf, [dst0 + e], v)
            return carry2

        lax.fori_loop(0, NFH, field, 0)

    def chunk(c, obuf, drain):
        r0 = base + c * C
        pltpu.sync_copy(cat_hbm.at[pl.ds(r0, C)], catb)
        cps0 = fire(0, gbuf0, sem0)
        cps1 = fire(1, gbuf1, sem1)
        cpc = pltpu.async_copy(
            cont_hbm.at[pl.ds(r0 * NCONT, C * NCONT)], cbuf, semc
        )
        if drain is not None:
            pltpu.make_async_copy(
                drain, out_hbm.at[pl.ds((r0 - C) * OUTW, C * OUTW)], semw
            ).wait()

        cpc.wait()
        for e in range(NCONT):
            v = plsc.load_gather(cbuf, [lane_cont + e])
            plsc.store_scatter(obuf, [lane_out + (NF * D + e)], v)

        for cp in cps0:
            cp.wait()
        merge(0, gbuf0, obuf)
        for cp in cps1:
            cp.wait()
        merge(1, gbuf1, obuf)

        pltpu.async_copy(obuf, out_hbm.at[pl.ds(r0 * OUTW, C * OUTW)], semw)

    chunk(0, obufa, None)
    chunk(1, obufb, obufa)

    def pair2(s2, carry):
        c0 = 2 * s2
        chunk(c0, obufa, obufb)
        chunk(c0 + 1, obufb, obufa)
        return carry

    lax.fori_loop(1, NCH // 2, pair2, 0)
    pltpu.make_async_copy(
        obufb, out_hbm.at[pl.ds((base + (NCH - 1) * C) * OUTW, C * OUTW)], semw
    ).wait()


def kernel(cont_data, cat_data, tables):
    out = _emb_gather(
        tables.reshape(TR, TW),
        cat_data,
        cont_data.reshape(B * NCONT),
    )
    return out.reshape(B, OUTW)


# speedup vs baseline: 1.1403x; 1.0128x over previous
import functools

import jax
import jax.numpy as jnp
from jax import lax
from jax.experimental import pallas as pl
from jax.experimental.pallas import tpu as pltpu
from jax.experimental.pallas import tpu_sc as plsc

NF = 26
V = 100000
D = 32
B = 16384
NCONT = 13
OUTW = NF * D + NCONT
TW = 128
TR = NF * V // 4
NFH = NF // 2

NC, NS = 2, 16
NW = NC * NS
RPW = B // NW
C = 16
NCH = RPW // C

_mesh = plsc.VectorSubcoreMesh(
    core_axis_name="c", subcore_axis_name="s", num_cores=NC, num_subcores=NS
)


@functools.partial(
    pl.kernel,
    out_type=jax.ShapeDtypeStruct((B * OUTW,), jnp.float32),
    mesh=_mesh,
    scratch_types=[
        pltpu.VMEM((C, NF), jnp.int32),
        pltpu.VMEM((C, NF), jnp.int32),
        pltpu.VMEM((NFH * C, TW), jnp.float32),
        pltpu.VMEM((NFH * C, TW), jnp.float32),
        pltpu.VMEM((C * NCONT,), jnp.float32),
        pltpu.VMEM((C * OUTW,), jnp.float32),
        pltpu.VMEM((C * OUTW,), jnp.float32),
        pltpu.SemaphoreType.DMA,
        pltpu.SemaphoreType.DMA,
        pltpu.SemaphoreType.DMA,
        pltpu.SemaphoreType.DMA,
        pltpu.SemaphoreType.DMA,
    ],
    compiler_params=pltpu.CompilerParams(needs_layout_passes=False),
)
def _emb_gather(table_hbm, cat_hbm, cont_hbm, out_hbm,
                catba, catbb, gbuf0, gbuf1, cbuf, obufa, obufb,
                sem0, sem1, semc, semw, semcat):
    wid = lax.axis_index("s") * NC + lax.axis_index("c")
    base = wid * RPW
    lane = jnp.arange(16, dtype=jnp.int32)
    lane_out = lane * OUTW
    lane_cont = lane * NCONT

    def quad_idx(catb, f):
        ids = plsc.load_gather(catb, [lane, jnp.full((16,), f, jnp.int32)])
        return (ids + f * V) >> 2

    def sub_off(catb, f):
        ids = plsc.load_gather(catb, [lane, jnp.full((16,), f, jnp.int32)])
        return ((ids + f * V) & 3) << 5

    def fire(catb, half, gbuf, sem):
        return [
            pltpu.async_copy(
                table_hbm.at[quad_idx(catb, half * NFH + f)],
                gbuf.at[pl.ds(f * C, C)],
                sem,
            )
            for f in range(NFH)
        ]

    def merge(catb, half, gbuf, obuf):
        def field(f, carry2):
            fa = half * NFH + f
            offv = sub_off(catb, fa)
            rows = f * C + lane
            dst0 = lane_out + fa * D
            for e in range(D):
                v = plsc.load_gather(gbuf, [rows, offv + e])
                plsc.store_scatter(obuf, [dst0 + e], v)
            return carry2

        lax.fori_loop(0, NFH, field, 0)

    def chunk(c, obuf, drain, cb_cur, cb_next, first=False):
        r0 = base + c * C
        if not first:
            pltpu.make_async_copy(
                cat_hbm.at[pl.ds(r0, C)], cb_cur, semcat
            ).wait()
        cps0 = fire(cb_cur, 0, gbuf0, sem0)
        cps1 = fire(cb_cur, 1, gbuf1, sem1)
        cpc = pltpu.async_copy(
            cont_hbm.at[pl.ds(r0 * NCONT, C * NCONT)], cbuf, semc
        )
        rn = jnp.minimum(r0 + C, B - C)
        pltpu.async_copy(cat_hbm.at[pl.ds(rn, C)], cb_next, semcat)
        if drain is not None:
            pltpu.make_async_copy(
                drain, out_hbm.at[pl.ds((r0 - C) * OUTW, C * OUTW)], semw
            ).wait()

        cpc.wait()
        for e in range(NCONT):
            v = plsc.load_gather(cbuf, [lane_cont + e])
            plsc.store_scatter(obuf, [lane_out + (NF * D + e)], v)

        for cp in cps0:
            cp.wait()
        merge(cb_cur, 0, gbuf0, obuf)
        for cp in cps1:
            cp.wait()
        merge(cb_cur, 1, gbuf1, obuf)

        pltpu.async_copy(obuf, out_hbm.at[pl.ds(r0 * OUTW, C * OUTW)], semw)

    pltpu.sync_copy(cat_hbm.at[pl.ds(base, C)], catba)
    chunk(0, obufa, None, catba, catbb, first=True)
    chunk(1, obufb, obufa, catbb, catba)

    def pair2(s2, carry):
        c0 = 2 * s2
        chunk(c0, obufa, obufb, catba, catbb)
        chunk(c0 + 1, obufb, obufa, catbb, catba)
        return carry

    lax.fori_loop(1, NCH // 2, pair2, 0)
    pltpu.make_async_copy(
        cat_hbm.at[pl.ds(B - C, C)], catba, semcat
    ).wait()
    pltpu.make_async_copy(
        obufb, out_hbm.at[pl.ds((base + (NCH - 1) * C) * OUTW, C * OUTW)], semw
    ).wait()


def kernel(cont_data, cat_data, tables):
    out = _emb_gather(
        tables.reshape(TR, TW),
        cat_data,
        cont_data.reshape(B * NCONT),
    )
    return out.reshape(B, OUTW)
